# baseline (speedup 1.0000x reference)
import functools

import jax
import jax.numpy as jnp
from jax import lax
from jax.experimental import pallas as pl
from jax.experimental.pallas import tpu as pltpu
from jax.experimental.pallas import tpu_sc as plsc

VOCAB = 1000000
EMBED_DIM = 32
BATCH = 16384

NUM_CORES = 2
NUM_SUBCORES = 16
NUM_WORKERS = NUM_CORES * NUM_SUBCORES
B_PER_W = BATCH // NUM_WORKERS
CHUNK = 128
N_CHUNKS = B_PER_W // CHUNK
LANES = 16
N_GROUPS = B_PER_W // LANES


def _glove_dot_sc(table, target_idx, context_idx):
  mesh = plsc.VectorSubcoreMesh(core_axis_name="c", subcore_axis_name="s")

  @functools.partial(
      pl.kernel,
      mesh=mesh,
      compiler_params=pltpu.CompilerParams(
          use_tc_tiling_on_sc=False, needs_layout_passes=False),
      out_type=jax.ShapeDtypeStruct((BATCH,), jnp.float32),
      scratch_types=[
          pltpu.VMEM((N_CHUNKS, CHUNK), jnp.int32),
          pltpu.VMEM((N_CHUNKS, CHUNK), jnp.int32),
          pltpu.VMEM((B_PER_W, EMBED_DIM), jnp.float32),
          pltpu.VMEM((B_PER_W, EMBED_DIM), jnp.float32),
          pltpu.VMEM((B_PER_W,), jnp.float32),
          pltpu.SemaphoreType.DMA,
          pltpu.SemaphoreType.DMA,
      ],
  )
  def k(table_hbm, tidx_hbm, cidx_hbm, out_hbm,
        tidx_v, cidx_v, trows_v, crows_v, out_v, sem_t, sem_c):
    wid = lax.axis_index("s") * NUM_CORES + lax.axis_index("c")
    base = wid * B_PER_W

    pltpu.sync_copy(tidx_hbm.at[pl.ds(wid * N_CHUNKS, N_CHUNKS)], tidx_v)
    pltpu.sync_copy(cidx_hbm.at[pl.ds(wid * N_CHUNKS, N_CHUNKS)], cidx_v)

    copies = []
    for kk in range(N_CHUNKS):
      copies.append(pltpu.async_copy(
          table_hbm.at[tidx_v.at[kk]],
          trows_v.at[pl.ds(kk * CHUNK, CHUNK)], sem_t))
      copies.append(pltpu.async_copy(
          table_hbm.at[cidx_v.at[kk]],
          crows_v.at[pl.ds(kk * CHUNK, CHUNK)], sem_c))
    for cp in copies:
      cp.wait()

    def group_body(g, carry):
      rows = g * LANES + lax.iota(jnp.int32, LANES)
      acc = jnp.zeros((LANES,), jnp.float32)
      for j in range(EMBED_DIM):
        colj = jnp.full((LANES,), j, jnp.int32)
        tv = plsc.load_gather(trows_v, [rows, colj])
        cv = plsc.load_gather(crows_v, [rows, colj])
        acc = acc + tv * cv
      out_v[pl.ds(g * LANES, LANES)] = acc
      return carry

    lax.fori_loop(0, N_GROUPS, group_body, 0)

    pltpu.sync_copy(out_v, out_hbm.at[pl.ds(base, B_PER_W)])

  tidx2d = target_idx.reshape(NUM_WORKERS * N_CHUNKS, CHUNK)
  cidx2d = context_idx.reshape(NUM_WORKERS * N_CHUNKS, CHUNK)
  return k(table, tidx2d, cidx2d)


def kernel(target, context, table):
  t = target.astype(jnp.int32)
  c = context.astype(jnp.int32)
  out = _glove_dot_sc(table, t, c)
  return out.reshape(BATCH, 1)
